# baseline (speedup 1.0000x reference)
import functools

import jax
import jax.numpy as jnp
import numpy as np
from jax import lax
from jax.experimental import pallas as pl
from jax.experimental.pallas import tpu as pltpu
from jax.experimental.pallas import tpu_sc as plsc

_IRREPS = [(128, 0), (64, 1), (32, 2), (16, 3)]
_N = 50000
_DIM = sum(mul * (2 * l + 1) for mul, l in _IRREPS)


def _generators(l):
    d = 2 * l + 1
    ms = np.arange(-l, l + 1)
    Lz = np.diag(ms).astype(complex)
    Lp = np.zeros((d, d), dtype=complex)
    for m in range(-l, l):
        Lp[m + 1 + l, m + l] = np.sqrt(l * (l + 1) - m * (m + 1))
    Lm = Lp.conj().T
    Lx = (Lp + Lm) / 2.0
    Ly = (Lp - Lm) / 2j
    return Lx, Ly, Lz


def _real_U(l):
    d = 2 * l + 1
    U = np.zeros((d, d), dtype=complex)
    U[l, l] = 1.0
    s = 1.0 / np.sqrt(2.0)
    for m in range(1, l + 1):
        U[l + m, l + m] = ((-1) ** m) * s
        U[l + m, l - m] = s
        U[l - m, l + m] = -1j * ((-1) ** m) * s
        U[l - m, l - m] = 1j * s
    return U


def _expm(A):
    w, V = np.linalg.eig(A.astype(complex))
    return np.real(V @ np.diag(np.exp(w)) @ np.linalg.inv(V))


_J_NNZ = {}
_KPAIR = {}
for _l in (1, 2, 3):
    _Lx, _Ly, _Lz = _generators(_l)
    _U = _real_U(_l)
    _Kx = np.real(-1j * (_U @ _Lx @ _U.conj().T))
    _Kz = np.real(-1j * (_U @ _Lz @ _U.conj().T))
    _Jm = _expm(np.pi * (_Kx + _Kz) / np.sqrt(2.0))
    _d = 2 * _l + 1
    _J_NNZ[_l] = [
        (i, j, float(_Jm[i, j]))
        for i in range(_d)
        for j in range(_d)
        if abs(_Jm[i, j]) > 1e-12
    ]
    _KPAIR[_l] = [float(_Kz[_l - m, _l + m]) for m in range(1, _l + 1)]

_NC = 2
_NS = 16
_NW = _NC * _NS

_BLOCK = 64
_UNITS = _N // 8
_U_PER_W = _UNITS // _NW
_U_REM = _UNITS - _U_PER_W * _NW
_AROWS = (_U_PER_W + 1) * 8
_NBLOCKS = -(-(_U_PER_W + 1) * 8 // _BLOCK)
_NBLOCKS += _NBLOCKS % 2


def _poly_sincos(x):
    x2 = x * x
    c = 1.0 + x2 * (-0.5 + x2 * (1.0 / 24 + x2 * (-1.0 / 720 + x2 * (1.0 / 40320))))
    s = x * (1.0 + x2 * (-1.0 / 6 + x2 * (1.0 / 120 + x2 * (-1.0 / 5040))))
    return c, s


def _trig_tables(theta):
    c1, s1 = _poly_sincos(theta)
    c2 = 2.0 * c1 * c1 - 1.0
    s2 = 2.0 * s1 * c1
    c3 = c2 * c1 - s2 * s1
    s3 = s2 * c1 + c2 * s1
    return [(c1, s1), (c2, s2), (c3, s3)]


def _dz_stage(v, trig, l):
    out = list(v)
    for m in range(1, l + 1):
        cm, sm = trig[m - 1]
        k = _KPAIR[l][m - 1]
        sgn = 1.0 if k > 0 else -1.0
        s = sm * sgn
        a, b = v[l - m], v[l + m]
        out[l - m] = cm * a + s * b
        out[l + m] = cm * b - s * a
    return out


def _j_stage(v, l):
    d = 2 * l + 1
    acc = [None] * d
    for i, j, val in _J_NNZ[l]:
        term = v[j] * val
        acc[i] = term if acc[i] is None else acc[i] + term
    return acc


_TS = 33


def _transform_set(buf, rvec, cols, l, ta, tb, tg):
    v = [plsc.load_gather(buf, [rvec, cols[j]]) for j in range(2 * l + 1)]
    v = _dz_stage(v, tg, l)
    v = _j_stage(v, l)
    v = _dz_stage(v, tb, l)
    v = _j_stage(v, l)
    v = _dz_stage(v, ta, l)
    for j in range(2 * l + 1):
        plsc.store_scatter(buf, [rvec, cols[j]], v[j])


def _body(inp_hbm, a_hbm, b_hbm, g_hbm, out_hbm,
          buf0, buf1, abuf, bbuf, gbuf, tbuf,
          isem0, isem1, osem0, osem1):
    cid = lax.axis_index("c")
    sid = lax.axis_index("s")
    wid = sid * _NC + cid
    u0 = wid * _U_PER_W + jnp.minimum(wid, _U_REM)
    cnt = _U_PER_W + jnp.where(wid < _U_REM, 1, 0)

    lane = lax.iota(jnp.int32, 16)
    bufs = (buf0, buf1)
    isems = (isem0, isem1)
    osems = (osem0, osem1)

    def block_row0(b):
        us = jnp.minimum(u0 + (_BLOCK // 8) * b, u0 + cnt - _BLOCK // 8)
        return us * 8

    astart = jnp.minimum(u0, _UNITS - _AROWS // 8) * 8
    pltpu.sync_copy(a_hbm.at[pl.ds(astart, _AROWS)], abuf)
    pltpu.sync_copy(b_hbm.at[pl.ds(astart, _AROWS)], bbuf)
    pltpu.sync_copy(g_hbm.at[pl.ds(astart, _AROWS)], gbuf)

    def compute_block(buf, row0):
        loc0 = row0 - astart

        def trig_body(g, c2):
            r0 = g * 16
            rows = lane + r0
            coeffs = []
            for angbuf in (abuf, bbuf, gbuf):
                for cm, sm in _trig_tables(angbuf[pl.ds(loc0 + r0, 16)]):
                    coeffs.append(cm)
                    coeffs.append(sm)
            for k, cv in enumerate(coeffs):
                plsc.store_scatter(tbuf, [rows * _TS + k], cv)
            return c2

        lax.fori_loop(0, _BLOCK // 16, trig_body, 0)

        @plsc.parallel_loop(0, _BLOCK, 1, unroll=8)
        def row_body(r):
            t0 = tbuf[pl.ds(r * _TS, 16)]
            t1 = tbuf[pl.ds(r * _TS + 16, 16)]

            def bc(k):
                src = t0 if k < 16 else t1
                return jnp.take_along_axis(
                    src, jnp.full((16,), k % 16, jnp.int32), axis=0)

            co = [bc(k) for k in range(18)]
            ta = [(co[0], co[1]), (co[2], co[3]), (co[4], co[5])]
            tb = [(co[6], co[7]), (co[8], co[9]), (co[10], co[11])]
            tg = [(co[12], co[13]), (co[14], co[15]), (co[16], co[17])]
            rvec = jnp.full((16,), 0, jnp.int32) + r
            lane3 = lane * 3
            lane5 = lane * 5
            lane7 = lane * 7
            for v in range(4):
                cols = [lane3 + (128 + 48 * v + j) for j in range(3)]
                _transform_set(buf, rvec, cols, 1, ta, tb, tg)
            for v in range(2):
                cols = [lane5 + (320 + 80 * v + j) for j in range(5)]
                _transform_set(buf, rvec, cols, 2, ta, tb, tg)
            cols = [lane7 + (480 + j) for j in range(7)]
            _transform_set(buf, rvec, cols, 3, ta, tb, tg)

    def wait_in(B):
        pltpu.make_async_copy(
            inp_hbm.at[pl.ds(0, _BLOCK)], bufs[B], isems[B]).wait()

    def wait_out(B):
        pltpu.make_async_copy(
            bufs[B], out_hbm.at[pl.ds(0, _BLOCK)], osems[B]).wait()

    pltpu.async_copy(inp_hbm.at[pl.ds(block_row0(0), _BLOCK)], buf0, isem0)

    def pair_body(p, carry):
        for sub in range(2):
            b = 2 * p + sub
            B = sub
            Bo = 1 - sub

            @pl.when(b + 1 < _NBLOCKS)
            def _prefetch():
                @pl.when(b >= 1)
                def _drain():
                    wait_out(Bo)

                pltpu.async_copy(
                    inp_hbm.at[pl.ds(block_row0(b + 1), _BLOCK)],
                    bufs[Bo], isems[Bo])

            wait_in(B)
            compute_block(bufs[B], block_row0(b))
            pltpu.async_copy(
                bufs[B], out_hbm.at[pl.ds(block_row0(b), _BLOCK)], osems[B])
        return carry

    lax.fori_loop(0, _NBLOCKS // 2, pair_body, 0)
    wait_out(0)
    wait_out(1)


@jax.jit
def _run(inp, alpha, beta, gamma):
    mesh = plsc.VectorSubcoreMesh(core_axis_name="c", subcore_axis_name="s")
    fn = pl.kernel(
        _body,
        out_type=jax.ShapeDtypeStruct((_N, _DIM), jnp.float32),
        mesh=mesh,
        scratch_types=[
            pltpu.VMEM((_BLOCK, _DIM), jnp.float32),
            pltpu.VMEM((_BLOCK, _DIM), jnp.float32),
            pltpu.VMEM((_AROWS,), jnp.float32),
            pltpu.VMEM((_AROWS,), jnp.float32),
            pltpu.VMEM((_AROWS,), jnp.float32),
            pltpu.VMEM((_BLOCK * _TS,), jnp.float32),
            pltpu.SemaphoreType.DMA,
            pltpu.SemaphoreType.DMA,
            pltpu.SemaphoreType.DMA,
            pltpu.SemaphoreType.DMA,
        ],
        compiler_params=pltpu.CompilerParams(needs_layout_passes=False),
    )
    return fn(inp, alpha, beta, gamma)


def kernel(input, alpha, beta, gamma):
    return _run(input, alpha, beta, gamma)


# speedup vs baseline: 1.0311x; 1.0311x over previous
import functools

import jax
import jax.numpy as jnp
import numpy as np
from jax import lax
from jax.experimental import pallas as pl
from jax.experimental.pallas import tpu as pltpu
from jax.experimental.pallas import tpu_sc as plsc

_IRREPS = [(128, 0), (64, 1), (32, 2), (16, 3)]
_N = 50000
_DIM = sum(mul * (2 * l + 1) for mul, l in _IRREPS)


def _generators(l):
    d = 2 * l + 1
    ms = np.arange(-l, l + 1)
    Lz = np.diag(ms).astype(complex)
    Lp = np.zeros((d, d), dtype=complex)
    for m in range(-l, l):
        Lp[m + 1 + l, m + l] = np.sqrt(l * (l + 1) - m * (m + 1))
    Lm = Lp.conj().T
    Lx = (Lp + Lm) / 2.0
    Ly = (Lp - Lm) / 2j
    return Lx, Ly, Lz


def _real_U(l):
    d = 2 * l + 1
    U = np.zeros((d, d), dtype=complex)
    U[l, l] = 1.0
    s = 1.0 / np.sqrt(2.0)
    for m in range(1, l + 1):
        U[l + m, l + m] = ((-1) ** m) * s
        U[l + m, l - m] = s
        U[l - m, l + m] = -1j * ((-1) ** m) * s
        U[l - m, l - m] = 1j * s
    return U


def _expm(A):
    w, V = np.linalg.eig(A.astype(complex))
    return np.real(V @ np.diag(np.exp(w)) @ np.linalg.inv(V))


_J_NNZ = {}
_KPAIR = {}
for _l in (1, 2, 3):
    _Lx, _Ly, _Lz = _generators(_l)
    _U = _real_U(_l)
    _Kx = np.real(-1j * (_U @ _Lx @ _U.conj().T))
    _Kz = np.real(-1j * (_U @ _Lz @ _U.conj().T))
    _Jm = _expm(np.pi * (_Kx + _Kz) / np.sqrt(2.0))
    _d = 2 * _l + 1
    _J_NNZ[_l] = [
        (i, j, float(_Jm[i, j]))
        for i in range(_d)
        for j in range(_d)
        if abs(_Jm[i, j]) > 1e-12
    ]
    _KPAIR[_l] = [float(_Kz[_l - m, _l + m]) for m in range(1, _l + 1)]

_NC = 2
_NS = 16
_NW = _NC * _NS

_BLOCK = 64
_UNITS = _N // 8
_U_PER_W = _UNITS // _NW
_U_REM = _UNITS - _U_PER_W * _NW
_AROWS = (_U_PER_W + 1) * 8
_NBLOCKS = -(-(_U_PER_W + 1) * 8 // _BLOCK)
_NBLOCKS += _NBLOCKS % 2


def _poly_sincos(x):
    x2 = x * x
    c = 1.0 + x2 * (-0.5 + x2 * (1.0 / 24 + x2 * (-1.0 / 720 + x2 * (1.0 / 40320))))
    s = x * (1.0 + x2 * (-1.0 / 6 + x2 * (1.0 / 120 + x2 * (-1.0 / 5040))))
    return c, s


def _trig_tables(theta):
    c1, s1 = _poly_sincos(theta)
    c2 = 2.0 * c1 * c1 - 1.0
    s2 = 2.0 * s1 * c1
    c3 = c2 * c1 - s2 * s1
    s3 = s2 * c1 + c2 * s1
    return [(c1, s1), (c2, s2), (c3, s3)]


def _dz_stage(v, trig, l):
    out = list(v)
    for m in range(1, l + 1):
        cm, sm = trig[m - 1]
        k = _KPAIR[l][m - 1]
        sgn = 1.0 if k > 0 else -1.0
        s = sm * sgn
        a, b = v[l - m], v[l + m]
        out[l - m] = cm * a + s * b
        out[l + m] = cm * b - s * a
    return out


def _j_stage(v, l):
    d = 2 * l + 1
    acc = [None] * d
    for i, j, val in _J_NNZ[l]:
        term = v[j] * val
        acc[i] = term if acc[i] is None else acc[i] + term
    return acc


_TS = 33


def _transform_set(buf, rvec, cols, l, ta, tb, tg):
    v = [plsc.load_gather(buf, [rvec, cols[j]]) for j in range(2 * l + 1)]
    v = _dz_stage(v, tg, l)
    v = _j_stage(v, l)
    v = _dz_stage(v, tb, l)
    v = _j_stage(v, l)
    v = _dz_stage(v, ta, l)
    for j in range(2 * l + 1):
        plsc.store_scatter(buf, [rvec, cols[j]], v[j])


def _body(inp_hbm, a_hbm, b_hbm, g_hbm, out_hbm,
          buf0, buf1, abuf, bbuf, gbuf, tbuf,
          isem0, isem1, osem0, osem1):
    cid = lax.axis_index("c")
    sid = lax.axis_index("s")
    wid = sid * _NC + cid
    u0 = wid * _U_PER_W + jnp.minimum(wid, _U_REM)
    cnt = _U_PER_W + jnp.where(wid < _U_REM, 1, 0)

    lane = lax.iota(jnp.int32, 16)
    bufs = (buf0, buf1)
    isems = (isem0, isem1)
    osems = (osem0, osem1)

    def block_row0(b):
        us = jnp.minimum(u0 + (_BLOCK // 8) * b, u0 + cnt - _BLOCK // 8)
        return us * 8

    astart = jnp.minimum(u0, _UNITS - _AROWS // 8) * 8
    pltpu.sync_copy(a_hbm.at[pl.ds(astart, _AROWS)], abuf)
    pltpu.sync_copy(b_hbm.at[pl.ds(astart, _AROWS)], bbuf)
    pltpu.sync_copy(g_hbm.at[pl.ds(astart, _AROWS)], gbuf)

    def compute_block(buf, row0):
        loc0 = row0 - astart

        def trig_body(g, c2):
            r0 = g * 16
            rows = lane + r0
            coeffs = []
            for angbuf in (abuf, bbuf, gbuf):
                for cm, sm in _trig_tables(angbuf[pl.ds(loc0 + r0, 16)]):
                    coeffs.append(cm)
                    coeffs.append(sm)
            for k, cv in enumerate(coeffs):
                plsc.store_scatter(tbuf, [rows * _TS + k], cv)
            return c2

        lax.fori_loop(0, _BLOCK // 16, trig_body, 0)

        @plsc.parallel_loop(0, _BLOCK, 1, unroll=4)
        def row_body(r):
            t0 = tbuf[pl.ds(r * _TS, 16)]
            t1 = tbuf[pl.ds(r * _TS + 16, 16)]

            def bc(k):
                src = t0 if k < 16 else t1
                return jnp.take_along_axis(
                    src, jnp.full((16,), k % 16, jnp.int32), axis=0)

            co = [bc(k) for k in range(18)]
            ta = [(co[0], co[1]), (co[2], co[3]), (co[4], co[5])]
            tb = [(co[6], co[7]), (co[8], co[9]), (co[10], co[11])]
            tg = [(co[12], co[13]), (co[14], co[15]), (co[16], co[17])]
            rvec = jnp.full((16,), 0, jnp.int32) + r
            lane3 = lane * 3
            lane5 = lane * 5
            lane7 = lane * 7
            for v in range(4):
                cols = [lane3 + (128 + 48 * v + j) for j in range(3)]
                _transform_set(buf, rvec, cols, 1, ta, tb, tg)
            for v in range(2):
                cols = [lane5 + (320 + 80 * v + j) for j in range(5)]
                _transform_set(buf, rvec, cols, 2, ta, tb, tg)
            cols = [lane7 + (480 + j) for j in range(7)]
            _transform_set(buf, rvec, cols, 3, ta, tb, tg)

    def wait_in(B):
        pltpu.make_async_copy(
            inp_hbm.at[pl.ds(0, _BLOCK)], bufs[B], isems[B]).wait()

    def wait_out(B):
        pltpu.make_async_copy(
            bufs[B], out_hbm.at[pl.ds(0, _BLOCK)], osems[B]).wait()

    pltpu.async_copy(inp_hbm.at[pl.ds(block_row0(0), _BLOCK)], buf0, isem0)

    def pair_body(p, carry):
        for sub in range(2):
            b = 2 * p + sub
            B = sub
            Bo = 1 - sub

            @pl.when(b + 1 < _NBLOCKS)
            def _prefetch():
                @pl.when(b >= 1)
                def _drain():
                    wait_out(Bo)

                pltpu.async_copy(
                    inp_hbm.at[pl.ds(block_row0(b + 1), _BLOCK)],
                    bufs[Bo], isems[Bo])

            wait_in(B)
            compute_block(bufs[B], block_row0(b))
            pltpu.async_copy(
                bufs[B], out_hbm.at[pl.ds(block_row0(b), _BLOCK)], osems[B])
        return carry

    lax.fori_loop(0, _NBLOCKS // 2, pair_body, 0)
    wait_out(0)
    wait_out(1)


@jax.jit
def _run(inp, alpha, beta, gamma):
    mesh = plsc.VectorSubcoreMesh(core_axis_name="c", subcore_axis_name="s")
    fn = pl.kernel(
        _body,
        out_type=jax.ShapeDtypeStruct((_N, _DIM), jnp.float32),
        mesh=mesh,
        scratch_types=[
            pltpu.VMEM((_BLOCK, _DIM), jnp.float32),
            pltpu.VMEM((_BLOCK, _DIM), jnp.float32),
            pltpu.VMEM((_AROWS,), jnp.float32),
            pltpu.VMEM((_AROWS,), jnp.float32),
            pltpu.VMEM((_AROWS,), jnp.float32),
            pltpu.VMEM((_BLOCK * _TS,), jnp.float32),
            pltpu.SemaphoreType.DMA,
            pltpu.SemaphoreType.DMA,
            pltpu.SemaphoreType.DMA,
            pltpu.SemaphoreType.DMA,
        ],
        compiler_params=pltpu.CompilerParams(needs_layout_passes=False),
    )
    return fn(inp, alpha, beta, gamma)


def kernel(input, alpha, beta, gamma):
    return _run(input, alpha, beta, gamma)


# speedup vs baseline: 1.1384x; 1.1041x over previous
import functools

import jax
import jax.numpy as jnp
import numpy as np
from jax import lax
from jax.experimental import pallas as pl
from jax.experimental.pallas import tpu as pltpu
from jax.experimental.pallas import tpu_sc as plsc

_IRREPS = [(128, 0), (64, 1), (32, 2), (16, 3)]
_N = 50000
_DIM = sum(mul * (2 * l + 1) for mul, l in _IRREPS)


def _generators(l):
    d = 2 * l + 1
    ms = np.arange(-l, l + 1)
    Lz = np.diag(ms).astype(complex)
    Lp = np.zeros((d, d), dtype=complex)
    for m in range(-l, l):
        Lp[m + 1 + l, m + l] = np.sqrt(l * (l + 1) - m * (m + 1))
    Lm = Lp.conj().T
    Lx = (Lp + Lm) / 2.0
    Ly = (Lp - Lm) / 2j
    return Lx, Ly, Lz


def _real_U(l):
    d = 2 * l + 1
    U = np.zeros((d, d), dtype=complex)
    U[l, l] = 1.0
    s = 1.0 / np.sqrt(2.0)
    for m in range(1, l + 1):
        U[l + m, l + m] = ((-1) ** m) * s
        U[l + m, l - m] = s
        U[l - m, l + m] = -1j * ((-1) ** m) * s
        U[l - m, l - m] = 1j * s
    return U


def _expm(A):
    w, V = np.linalg.eig(A.astype(complex))
    return np.real(V @ np.diag(np.exp(w)) @ np.linalg.inv(V))


_J_NNZ = {}
_KPAIR = {}
for _l in (1, 2, 3):
    _Lx, _Ly, _Lz = _generators(_l)
    _U = _real_U(_l)
    _Kx = np.real(-1j * (_U @ _Lx @ _U.conj().T))
    _Kz = np.real(-1j * (_U @ _Lz @ _U.conj().T))
    _Jm = _expm(np.pi * (_Kx + _Kz) / np.sqrt(2.0))
    _d = 2 * _l + 1
    _J_NNZ[_l] = [
        (i, j, float(_Jm[i, j]))
        for i in range(_d)
        for j in range(_d)
        if abs(_Jm[i, j]) > 1e-12
    ]
    _KPAIR[_l] = [float(_Kz[_l - m, _l + m]) for m in range(1, _l + 1)]

_NC = 2
_NS = 16
_NW = _NC * _NS

_BLOCK = 48
_NBUF = 3
_UNITS = _N // 8
_U_PER_W = _UNITS // _NW
_U_REM = _UNITS - _U_PER_W * _NW
_AROWS = (_U_PER_W + 1) * 8
_NBLOCKS = -(-(_U_PER_W + 1) * 8 // _BLOCK)
_NBLOCKS += (-_NBLOCKS) % _NBUF


def _poly_sincos(x):
    x2 = x * x
    c = 1.0 + x2 * (-0.5 + x2 * (1.0 / 24 + x2 * (-1.0 / 720 + x2 * (1.0 / 40320))))
    s = x * (1.0 + x2 * (-1.0 / 6 + x2 * (1.0 / 120 + x2 * (-1.0 / 5040))))
    return c, s


def _trig_tables(theta):
    c1, s1 = _poly_sincos(theta)
    c2 = 2.0 * c1 * c1 - 1.0
    s2 = 2.0 * s1 * c1
    c3 = c2 * c1 - s2 * s1
    s3 = s2 * c1 + c2 * s1
    return [(c1, s1), (c2, s2), (c3, s3)]


def _dz_stage(v, trig, l):
    out = list(v)
    for m in range(1, l + 1):
        cm, sm = trig[m - 1]
        k = _KPAIR[l][m - 1]
        sgn = 1.0 if k > 0 else -1.0
        s = sm * sgn
        a, b = v[l - m], v[l + m]
        out[l - m] = cm * a + s * b
        out[l + m] = cm * b - s * a
    return out


def _j_stage(v, l):
    d = 2 * l + 1
    acc = [None] * d
    for i, j, val in _J_NNZ[l]:
        term = v[j] * val
        acc[i] = term if acc[i] is None else acc[i] + term
    return acc


_TS = 33


def _transform_set(buf, rvec, cols, l, ta, tb, tg):
    v = [plsc.load_gather(buf, [rvec, cols[j]]) for j in range(2 * l + 1)]
    v = _dz_stage(v, tg, l)
    v = _j_stage(v, l)
    v = _dz_stage(v, tb, l)
    v = _j_stage(v, l)
    v = _dz_stage(v, ta, l)
    for j in range(2 * l + 1):
        plsc.store_scatter(buf, [rvec, cols[j]], v[j])


def _body(inp_hbm, a_hbm, b_hbm, g_hbm, out_hbm,
          buf0, buf1, buf2, abuf, bbuf, gbuf, tbuf,
          isem0, isem1, isem2, osem0, osem1, osem2):
    cid = lax.axis_index("c")
    sid = lax.axis_index("s")
    wid = sid * _NC + cid
    u0 = wid * _U_PER_W + jnp.minimum(wid, _U_REM)
    cnt = _U_PER_W + jnp.where(wid < _U_REM, 1, 0)

    lane = lax.iota(jnp.int32, 16)
    bufs = (buf0, buf1, buf2)
    isems = (isem0, isem1, isem2)
    osems = (osem0, osem1, osem2)

    def block_row0(b):
        us = jnp.minimum(u0 + (_BLOCK // 8) * b, u0 + cnt - _BLOCK // 8)
        return us * 8

    astart = jnp.minimum(u0, _UNITS - _AROWS // 8) * 8
    pltpu.sync_copy(a_hbm.at[pl.ds(astart, _AROWS)], abuf)
    pltpu.sync_copy(b_hbm.at[pl.ds(astart, _AROWS)], bbuf)
    pltpu.sync_copy(g_hbm.at[pl.ds(astart, _AROWS)], gbuf)

    def compute_block(buf, row0):
        loc0 = row0 - astart

        def trig_body(g, c2):
            r0 = g * 16
            rows = lane + r0
            coeffs = []
            for angbuf in (abuf, bbuf, gbuf):
                for cm, sm in _trig_tables(angbuf[pl.ds(loc0 + r0, 16)]):
                    coeffs.append(cm)
                    coeffs.append(sm)
            for k, cv in enumerate(coeffs):
                plsc.store_scatter(tbuf, [rows * _TS + k], cv)
            return c2

        lax.fori_loop(0, _BLOCK // 16, trig_body, 0)

        @plsc.parallel_loop(0, _BLOCK, 1, unroll=4)
        def row_body(r):
            t0 = tbuf[pl.ds(r * _TS, 16)]
            t1 = tbuf[pl.ds(r * _TS + 16, 16)]

            def bc(k):
                src = t0 if k < 16 else t1
                return jnp.take_along_axis(
                    src, jnp.full((16,), k % 16, jnp.int32), axis=0)

            co = [bc(k) for k in range(18)]
            ta = [(co[0], co[1]), (co[2], co[3]), (co[4], co[5])]
            tb = [(co[6], co[7]), (co[8], co[9]), (co[10], co[11])]
            tg = [(co[12], co[13]), (co[14], co[15]), (co[16], co[17])]
            rvec = jnp.full((16,), 0, jnp.int32) + r
            lane3 = lane * 3
            lane5 = lane * 5
            lane7 = lane * 7
            for v in range(4):
                cols = [lane3 + (128 + 48 * v + j) for j in range(3)]
                _transform_set(buf, rvec, cols, 1, ta, tb, tg)
            for v in range(2):
                cols = [lane5 + (320 + 80 * v + j) for j in range(5)]
                _transform_set(buf, rvec, cols, 2, ta, tb, tg)
            cols = [lane7 + (480 + j) for j in range(7)]
            _transform_set(buf, rvec, cols, 3, ta, tb, tg)

    def wait_in(B):
        pltpu.make_async_copy(
            inp_hbm.at[pl.ds(0, _BLOCK)], bufs[B], isems[B]).wait()

    def wait_out(B):
        pltpu.make_async_copy(
            bufs[B], out_hbm.at[pl.ds(0, _BLOCK)], osems[B]).wait()

    def start_in(b, B):
        pltpu.async_copy(
            inp_hbm.at[pl.ds(block_row0(b), _BLOCK)], bufs[B], isems[B])

    start_in(0, 0)
    start_in(1, 1)

    def trip_body(p, carry):
        for sub in range(_NBUF):
            b = _NBUF * p + sub
            B = sub
            Bp = (sub + 2) % _NBUF

            wait_in(B)
            compute_block(bufs[B], block_row0(b))
            pltpu.async_copy(
                bufs[B], out_hbm.at[pl.ds(block_row0(b), _BLOCK)], osems[B])

            @pl.when(b >= 1)
            def _drain():
                wait_out(Bp)

            start_in(b + 2, Bp)
        return carry

    lax.fori_loop(0, _NBLOCKS // _NBUF, trip_body, 0)
    wait_in(_NBLOCKS % _NBUF)
    wait_in((_NBLOCKS + 1) % _NBUF)
    wait_out((_NBLOCKS - 1) % _NBUF)


@jax.jit
def _run(inp, alpha, beta, gamma):
    mesh = plsc.VectorSubcoreMesh(core_axis_name="c", subcore_axis_name="s")
    fn = pl.kernel(
        _body,
        out_type=jax.ShapeDtypeStruct((_N, _DIM), jnp.float32),
        mesh=mesh,
        scratch_types=[
            pltpu.VMEM((_BLOCK, _DIM), jnp.float32),
            pltpu.VMEM((_BLOCK, _DIM), jnp.float32),
            pltpu.VMEM((_BLOCK, _DIM), jnp.float32),
            pltpu.VMEM((_AROWS,), jnp.float32),
            pltpu.VMEM((_AROWS,), jnp.float32),
            pltpu.VMEM((_AROWS,), jnp.float32),
            pltpu.VMEM((_BLOCK * _TS,), jnp.float32),
            pltpu.SemaphoreType.DMA,
            pltpu.SemaphoreType.DMA,
            pltpu.SemaphoreType.DMA,
            pltpu.SemaphoreType.DMA,
            pltpu.SemaphoreType.DMA,
            pltpu.SemaphoreType.DMA,
        ],
        compiler_params=pltpu.CompilerParams(needs_layout_passes=False),
    )
    return fn(inp, alpha, beta, gamma)


def kernel(input, alpha, beta, gamma):
    return _run(input, alpha, beta, gamma)
